# speedup vs baseline: 2.9548x; 2.9548x over previous
import functools

import jax
import jax.numpy as jnp
from jax import lax
from jax.experimental import pallas as pl
from jax.experimental.pallas import tpu as pltpu
from jax.experimental.pallas import tpu_sc as plsc

N = 10000
E = 320000
D = 128
B = 4096
R_FD = 2.0
T_FD = 1.0

NC = 2
NS = 16
NW = NC * NS

NPAD = 10240
ROWS_PER_TILE = NPAD // NS
EPW = 10240
EPAD = EPW * NW
CHUNK = 128
NCHUNK = EPW // CHUNK
PPW = B // NW
STG = 128

RB = 1024
DB = 512


def _expmap0_proj(u):
    nrm = jnp.maximum(jnp.sqrt(jnp.sum(u * u, axis=1, keepdims=True)), 1e-6)
    x = jnp.tanh(nrm) * u / nrm
    n2 = jnp.maximum(jnp.sqrt(jnp.sum(x * x, axis=1, keepdims=True)), 1e-6)
    return x * jnp.minimum(1.0, (1.0 - 1e-5) / n2)


def _logmap0(x):
    nrm = jnp.maximum(jnp.sqrt(jnp.sum(x * x, axis=1, keepdims=True)), 1e-6)
    p = jnp.clip(nrm, 1e-6, 1.0 - 1e-5)
    return 0.5 * jnp.log((1.0 + p) / (1.0 - p)) * x / nrm


def _stage1_body(x_ref, w_ref, b_ref, o_ref):
    xh = _expmap0_proj(x_ref[...])
    h = _logmap0(xh)
    o_ref[...] = (
        jnp.dot(h, w_ref[...], preferred_element_type=jnp.float32) + b_ref[...]
    )


def _stage2_body(part_ref, degp_ref, w_ref, b_ref, h1_ref, m2_ref):
    deg = jnp.maximum(degp_ref[0] + degp_ref[1], 1.0)
    agg = (part_ref[0] + part_ref[1]) / deg
    agg = jnp.maximum(agg, 0.0)
    h1 = _expmap0_proj(agg)
    h1_ref[...] = h1
    h = _logmap0(h1)
    m2_ref[...] = (
        jnp.dot(h, w_ref[...], preferred_element_type=jnp.float32) + b_ref[...]
    )


def _stage3_body(part_ref, degp_ref, h2_ref):
    deg = jnp.maximum(degp_ref[0] + degp_ref[1], 1.0)
    agg = (part_ref[0] + part_ref[1]) / deg
    h2_ref[...] = _expmap0_proj(agg)


def _stage4_body(l1_ref, r1_ref, l2_ref, r2_ref, o_ref):
    a = l1_ref[...] - r1_ref[...]
    b = l2_ref[...] - r2_ref[...]
    dist = jnp.sum(a * a + b * b, axis=1, keepdims=True)
    o_ref[...] = 1.0 / (jnp.exp((dist - R_FD) / T_FD) + 1.0)


def _stage1(xpad, W1, b1):
    return pl.pallas_call(
        _stage1_body,
        grid=(NPAD // RB,),
        in_specs=[
            pl.BlockSpec((RB, D), lambda i: (i, 0)),
            pl.BlockSpec((D, D), lambda i: (0, 0)),
            pl.BlockSpec((1, D), lambda i: (0, 0)),
        ],
        out_specs=pl.BlockSpec((RB, D), lambda i: (i, 0)),
        out_shape=jax.ShapeDtypeStruct((NPAD, D), jnp.float32),
    )(xpad, W1, b1)


def _stage2(part1, degp3, W2, b2):
    return pl.pallas_call(
        _stage2_body,
        grid=(NPAD // RB,),
        in_specs=[
            pl.BlockSpec((NC, RB, D), lambda i: (0, i, 0)),
            pl.BlockSpec((NC, RB, 1), lambda i: (0, i, 0)),
            pl.BlockSpec((D, D), lambda i: (0, 0)),
            pl.BlockSpec((1, D), lambda i: (0, 0)),
        ],
        out_specs=[
            pl.BlockSpec((RB, D), lambda i: (i, 0)),
            pl.BlockSpec((RB, D), lambda i: (i, 0)),
        ],
        out_shape=[
            jax.ShapeDtypeStruct((NPAD, D), jnp.float32),
            jax.ShapeDtypeStruct((NPAD, D), jnp.float32),
        ],
    )(part1, degp3, W2, b2)


def _stage3(part2, degp3):
    return pl.pallas_call(
        _stage3_body,
        grid=(NPAD // RB,),
        in_specs=[
            pl.BlockSpec((NC, RB, D), lambda i: (0, i, 0)),
            pl.BlockSpec((NC, RB, 1), lambda i: (0, i, 0)),
        ],
        out_specs=pl.BlockSpec((RB, D), lambda i: (i, 0)),
        out_shape=jax.ShapeDtypeStruct((NPAD, D), jnp.float32),
    )(part2, degp3)


def _stage4(L1, R1, L2, R2):
    return pl.pallas_call(
        _stage4_body,
        grid=(B // DB,),
        in_specs=[pl.BlockSpec((DB, D), lambda i: (i, 0))] * 4,
        out_specs=pl.BlockSpec((DB, 1), lambda i: (i, 0)),
        out_shape=jax.ShapeDtypeStruct((B, 1), jnp.float32),
    )(L1, R1, L2, R2)


def _sc_mesh():
    return plsc.VectorSubcoreMesh(core_axis_name="c", subcore_axis_name="s")


def _agg_deg_body(m, src, dst, zrows, dzer, ones, part, degp,
                  srcv, dstv, rows, sem, acc, onesv, dstg, dega):
    c = lax.axis_index("c")
    s = lax.axis_index("s")
    wid = s * NC + c
    r0 = s * ROWS_PER_TILE
    pltpu.sync_copy(zrows, rows)
    for t in range(ROWS_PER_TILE // STG):
        pltpu.sync_copy(rows, acc.at[pl.ds(r0 + t * STG, STG), :])
    pltpu.sync_copy(dzer, dstg)
    pltpu.sync_copy(dstg, dega.at[pl.ds(r0, ROWS_PER_TILE)])
    pltpu.sync_copy(ones, onesv)
    plsc.subcore_barrier()

    e0 = wid * EPW

    def step(j, carry):
        off = e0 + j * CHUNK
        pltpu.sync_copy(src.at[pl.ds(off, CHUNK)], srcv)
        pltpu.sync_copy(dst.at[pl.ds(off, CHUNK)], dstv)
        pltpu.async_copy(m.at[srcv], rows, sem).wait()
        pltpu.sync_copy(rows, acc.at[dstv], add=True)
        pltpu.sync_copy(onesv, dega.at[dstv], add=True)
        return carry

    lax.fori_loop(0, NCHUNK, step, 0)
    plsc.subcore_barrier()
    for t in range(ROWS_PER_TILE // STG):
        sl = pl.ds(r0 + t * STG, STG)
        pltpu.sync_copy(acc.at[sl, :], rows)
        pltpu.sync_copy(rows, part.at[c, sl, :])
    pltpu.sync_copy(dega.at[pl.ds(r0, ROWS_PER_TILE)], dstg)
    pltpu.sync_copy(dstg, degp.at[c, pl.ds(r0, ROWS_PER_TILE)])


def _agg_body(m, src, dst, zrows, part, srcv, dstv, rows, sem, acc):
    c = lax.axis_index("c")
    s = lax.axis_index("s")
    wid = s * NC + c
    r0 = s * ROWS_PER_TILE
    pltpu.sync_copy(zrows, rows)
    for t in range(ROWS_PER_TILE // STG):
        pltpu.sync_copy(rows, acc.at[pl.ds(r0 + t * STG, STG), :])
    plsc.subcore_barrier()

    e0 = wid * EPW

    def step(j, carry):
        off = e0 + j * CHUNK
        pltpu.sync_copy(src.at[pl.ds(off, CHUNK)], srcv)
        pltpu.sync_copy(dst.at[pl.ds(off, CHUNK)], dstv)
        pltpu.async_copy(m.at[srcv], rows, sem).wait()
        pltpu.sync_copy(rows, acc.at[dstv], add=True)
        return carry

    lax.fori_loop(0, NCHUNK, step, 0)
    plsc.subcore_barrier()
    for t in range(ROWS_PER_TILE // STG):
        sl = pl.ds(r0 + t * STG, STG)
        pltpu.sync_copy(acc.at[sl, :], rows)
        pltpu.sync_copy(rows, part.at[c, sl, :])


def _agg_deg(m, src, dst, zrows, dzer, ones):
    return pl.kernel(
        _agg_deg_body,
        mesh=_sc_mesh(),
        out_type=[
            jax.ShapeDtypeStruct((NC, NPAD, D), jnp.float32),
            jax.ShapeDtypeStruct((NC, NPAD), jnp.float32),
        ],
        scratch_types=[
            pltpu.VMEM((CHUNK,), jnp.int32),
            pltpu.VMEM((CHUNK,), jnp.int32),
            pltpu.VMEM((CHUNK, D), jnp.float32),
            pltpu.SemaphoreType.DMA,
            pltpu.VMEM_SHARED((NPAD, D), jnp.float32),
            pltpu.VMEM((CHUNK,), jnp.float32),
            pltpu.VMEM((ROWS_PER_TILE,), jnp.float32),
            pltpu.VMEM_SHARED((NPAD,), jnp.float32),
        ],
    )(m, src, dst, zrows, dzer, ones)


def _agg(m, src, dst, zrows):
    return pl.kernel(
        _agg_body,
        mesh=_sc_mesh(),
        out_type=jax.ShapeDtypeStruct((NC, NPAD, D), jnp.float32),
        scratch_types=[
            pltpu.VMEM((CHUNK,), jnp.int32),
            pltpu.VMEM((CHUNK,), jnp.int32),
            pltpu.VMEM((CHUNK, D), jnp.float32),
            pltpu.SemaphoreType.DMA,
            pltpu.VMEM_SHARED((NPAD, D), jnp.float32),
        ],
    )(m, src, dst, zrows)


def _pair_gather_body(h1, h2, il, ir, L1, L2, R1, R2, idxv, rows, sem):
    c = lax.axis_index("c")
    s = lax.axis_index("s")
    wid = s * NC + c
    sl = pl.ds(wid * PPW, PPW)
    pltpu.sync_copy(il.at[sl], idxv)
    pltpu.async_copy(h1.at[idxv], rows, sem).wait()
    pltpu.sync_copy(rows, L1.at[sl, :])
    pltpu.async_copy(h2.at[idxv], rows, sem).wait()
    pltpu.sync_copy(rows, L2.at[sl, :])
    pltpu.sync_copy(ir.at[sl], idxv)
    pltpu.async_copy(h1.at[idxv], rows, sem).wait()
    pltpu.sync_copy(rows, R1.at[sl, :])
    pltpu.async_copy(h2.at[idxv], rows, sem).wait()
    pltpu.sync_copy(rows, R2.at[sl, :])


def _pair_gather(h1, h2, il, ir):
    return pl.kernel(
        _pair_gather_body,
        mesh=_sc_mesh(),
        out_type=[jax.ShapeDtypeStruct((B, D), jnp.float32)] * 4,
        scratch_types=[
            pltpu.VMEM((PPW,), jnp.int32),
            pltpu.VMEM((PPW, D), jnp.float32),
            pltpu.SemaphoreType.DMA,
        ],
    )(h1, h2, il, ir)


@jax.jit
def kernel(x, adj, idx, W1, b1, W2, b2):
    src = adj[0].astype(jnp.int32)
    dst = adj[1].astype(jnp.int32)
    il = idx[:, 0].astype(jnp.int32)
    ir = idx[:, 1].astype(jnp.int32)
    pad_e = EPAD - E
    src = jnp.concatenate([src, jnp.zeros((pad_e,), jnp.int32)])
    dst = jnp.concatenate([dst, jnp.full((pad_e,), NPAD - 1, jnp.int32)])
    xpad = jnp.pad(x, ((0, NPAD - N), (0, 0)))
    zrows = jnp.zeros((STG, D), jnp.float32)
    dzer = jnp.zeros((ROWS_PER_TILE,), jnp.float32)
    ones = jnp.ones((CHUNK,), jnp.float32)

    m1 = _stage1(xpad, W1, b1.reshape(1, D))
    part1, degp = _agg_deg(m1, src, dst, zrows, dzer, ones)
    degp3 = degp.reshape(NC, NPAD, 1)
    h1, m2 = _stage2(part1, degp3, W2, b2.reshape(1, D))
    part2 = _agg(m2, src, dst, zrows)
    h2 = _stage3(part2, degp3)
    L1, L2, R1, R2 = _pair_gather(h1, h2, il, ir)
    probs = _stage4(L1, R1, L2, R2)
    return probs.reshape(B)
